# baseline (speedup 1.0000x reference)
import functools

import jax
import jax.numpy as jnp
from jax import lax
from jax.experimental import pallas as pl
from jax.experimental.pallas import tpu as pltpu
from jax.experimental.pallas import tpu_sc as plsc

N = 2048
D = 768
F = 1536
E = 8
TB = 256
G = 24
P = G * TB
NW = 32
CHUNK = N // NW


def _gate_body(x_ref, wg1_ref, bg1_ref, wg2_ref, bg2_ref,
               pos1_ref, pos2_ref, w1b_ref, w2b_ref, eid_ref):
    xb = x_ref[...]
    g1 = jnp.maximum(
        jnp.dot(xb, wg1_ref[...], preferred_element_type=jnp.float32)
        + bg1_ref[...], 0.0)
    logits = (jnp.dot(g1, wg2_ref[...], preferred_element_type=jnp.float32)
              + bg2_ref[...])
    m = jnp.max(logits, axis=-1, keepdims=True)
    ex = jnp.exp(logits - m)
    p = ex / jnp.sum(ex, axis=-1, keepdims=True)
    iota_e = jax.lax.broadcasted_iota(jnp.int32, (N, E), 1)
    w1 = jnp.max(p, axis=-1, keepdims=True)
    i1 = jnp.min(jnp.where(p >= w1, iota_e, E + 1), axis=-1, keepdims=True)
    pm = jnp.where(iota_e == i1, -jnp.inf, p)
    w2 = jnp.max(pm, axis=-1, keepdims=True)
    i2 = jnp.min(jnp.where(pm >= w2, iota_e, E + 1), axis=-1, keepdims=True)
    a = jnp.exp(w2 - w1)
    c1 = 1.0 / (1.0 + a)
    c2 = a * c1

    oh1 = (iota_e == i1).astype(jnp.float32)
    oh2 = (iota_e == i2).astype(jnp.float32)
    slot = oh1 + oh2
    rr = jax.lax.broadcasted_iota(jnp.int32, (N, N), 0)
    cc = jax.lax.broadcasted_iota(jnp.int32, (N, N), 1)
    lt = (cc < rr).astype(jnp.float32)
    rank = jnp.dot(lt, slot, preferred_element_type=jnp.float32)
    counts = jnp.sum(slot, axis=0, keepdims=True)
    cblk = jnp.floor((counts + (TB - 1.0)) * (1.0 / TB))
    ei = jax.lax.broadcasted_iota(jnp.int32, (E, E), 0)
    ej = jax.lax.broadcasted_iota(jnp.int32, (E, E), 1)
    ltE = (ei < ej).astype(jnp.float32)
    off_al = jnp.dot(cblk, ltE, preferred_element_type=jnp.float32) * TB
    dest = off_al + rank
    pos1_ref[...] = jnp.sum(oh1 * dest, axis=-1, keepdims=True).astype(jnp.int32)
    pos2_ref[...] = jnp.sum(oh2 * dest, axis=-1, keepdims=True).astype(jnp.int32)
    w1b_ref[...] = jnp.broadcast_to(c1, (N, 128))
    w2b_ref[...] = jnp.broadcast_to(c2, (N, 128))
    gi = jax.lax.broadcasted_iota(jnp.int32, (G, E), 0).astype(jnp.float32) * TB
    started = (jnp.broadcast_to(off_al, (G, E)) <= gi).astype(jnp.float32)
    eid_ref[...] = (jnp.sum(started, axis=-1, keepdims=True) - 1.0).astype(jnp.int32)


def _gate(x_flat, Wg1, bg1, Wg2, bg2):
    return pl.pallas_call(
        _gate_body,
        in_specs=[
            pl.BlockSpec((N, D), lambda: (0, 0)),
            pl.BlockSpec((D, D // 2), lambda: (0, 0)),
            pl.BlockSpec((1, D // 2), lambda: (0, 0)),
            pl.BlockSpec((D // 2, E), lambda: (0, 0)),
            pl.BlockSpec((1, E), lambda: (0, 0)),
        ],
        out_specs=[
            pl.BlockSpec((N, 1), lambda: (0, 0)),
            pl.BlockSpec((N, 1), lambda: (0, 0)),
            pl.BlockSpec((N, 128), lambda: (0, 0)),
            pl.BlockSpec((N, 128), lambda: (0, 0)),
            pl.BlockSpec((G, 1), lambda: (0, 0)),
        ],
        out_shape=[
            jax.ShapeDtypeStruct((N, 1), jnp.int32),
            jax.ShapeDtypeStruct((N, 1), jnp.int32),
            jax.ShapeDtypeStruct((N, 128), jnp.float32),
            jax.ShapeDtypeStruct((N, 128), jnp.float32),
            jax.ShapeDtypeStruct((G, 1), jnp.int32),
        ],
    )(x_flat, Wg1, bg1.reshape(1, -1), Wg2, bg2.reshape(1, -1))


_SC_MESH = plsc.VectorSubcoreMesh(core_axis_name="c", subcore_axis_name="s")


@functools.partial(
    pl.kernel,
    out_type=[
        jax.ShapeDtypeStruct((P, D), jnp.float32),
        jax.ShapeDtypeStruct((P, 128), jnp.float32),
    ],
    mesh=_SC_MESH,
    scratch_types=[
        pltpu.VMEM((CHUNK, D), jnp.float32),
        pltpu.VMEM((CHUNK, 128), jnp.float32),
        pltpu.VMEM((CHUNK, 128), jnp.float32),
        pltpu.VMEM((CHUNK,), jnp.int32),
        pltpu.VMEM((CHUNK,), jnp.int32),
        pltpu.SemaphoreType.DMA,
    ],
)
def _sc_scatter(x_hbm, pos1_hbm, pos2_hbm, w1b_hbm, w2b_hbm, xs_hbm, ws_hbm,
                xbuf, w1v, w2v, i1v, i2v, sem):
    wid = lax.axis_index("s") * 2 + lax.axis_index("c")
    base = wid * CHUNK
    pltpu.sync_copy(pos1_hbm.at[pl.ds(base, CHUNK)], i1v)
    pltpu.sync_copy(pos2_hbm.at[pl.ds(base, CHUNK)], i2v)
    pltpu.sync_copy(w1b_hbm.at[pl.ds(base, CHUNK)], w1v)
    pltpu.sync_copy(w2b_hbm.at[pl.ds(base, CHUNK)], w2v)
    pltpu.sync_copy(x_hbm.at[pl.ds(base, CHUNK)], xbuf)
    c1 = pltpu.async_copy(xbuf, xs_hbm.at[i1v], sem)
    c2 = pltpu.async_copy(xbuf, xs_hbm.at[i2v], sem)
    c3 = pltpu.async_copy(w1v, ws_hbm.at[i1v], sem)
    c4 = pltpu.async_copy(w2v, ws_hbm.at[i2v], sem)
    c1.wait()
    c2.wait()
    c3.wait()
    c4.wait()


def _ffn_body(eid_ref, x_ref, ws_ref, w1_ref, b1_ref, w2_ref, b2_ref, o_ref):
    del eid_ref
    xb = x_ref[...].astype(jnp.bfloat16)
    h = (jnp.dot(xb, w1_ref[0], preferred_element_type=jnp.float32)
         + b1_ref[0])
    h = 0.5 * h * (1.0 + jax.lax.erf(h * (2.0 ** -0.5)))
    o = (jnp.dot(h.astype(jnp.bfloat16), w2_ref[0],
                 preferred_element_type=jnp.float32) + b2_ref[0])
    o_ref[...] = o * ws_ref[:, 0:1]


def _ffn(eid, xs, ws, W1, b1, W2, b2):
    grid_spec = pltpu.PrefetchScalarGridSpec(
        num_scalar_prefetch=1,
        grid=(G,),
        in_specs=[
            pl.BlockSpec((TB, D), lambda g, eid: (g, 0)),
            pl.BlockSpec((TB, 128), lambda g, eid: (g, 0)),
            pl.BlockSpec((1, D, F), lambda g, eid: (eid[g], 0, 0)),
            pl.BlockSpec((1, 1, F), lambda g, eid: (eid[g], 0, 0)),
            pl.BlockSpec((1, F, D), lambda g, eid: (eid[g], 0, 0)),
            pl.BlockSpec((1, 1, D), lambda g, eid: (eid[g], 0, 0)),
        ],
        out_specs=pl.BlockSpec((TB, D), lambda g, eid: (g, 0)),
    )
    return pl.pallas_call(
        _ffn_body,
        grid_spec=grid_spec,
        out_shape=jax.ShapeDtypeStruct((P, D), jnp.float32),
    )(eid, xs, ws, W1.astype(jnp.bfloat16), b1.reshape(E, 1, F),
      W2.astype(jnp.bfloat16), b2.reshape(E, 1, D))


@functools.partial(
    pl.kernel,
    out_type=jax.ShapeDtypeStruct((N, D), jnp.float32),
    mesh=_SC_MESH,
    scratch_types=[
        pltpu.VMEM((CHUNK, D), jnp.float32),
        pltpu.VMEM((CHUNK, D), jnp.float32),
        pltpu.VMEM((CHUNK,), jnp.int32),
        pltpu.VMEM((CHUNK,), jnp.int32),
        pltpu.SemaphoreType.DMA,
    ],
)
def _sc_combine(y_hbm, pos1_hbm, pos2_hbm, out_hbm, y1, y2, i1v, i2v, sem):
    wid = lax.axis_index("s") * 2 + lax.axis_index("c")
    base = wid * CHUNK
    pltpu.sync_copy(pos1_hbm.at[pl.ds(base, CHUNK)], i1v)
    pltpu.sync_copy(pos2_hbm.at[pl.ds(base, CHUNK)], i2v)
    c1 = pltpu.async_copy(y_hbm.at[i1v], y1, sem)
    c2 = pltpu.async_copy(y_hbm.at[i2v], y2, sem)
    c1.wait()
    c2.wait()

    def row(i, _):
        for j in range(D // 16):
            sl = pl.ds(j * 16, 16)
            y1[i, sl] = y1[i, sl] + y2[i, sl]
        return 0

    lax.fori_loop(0, CHUNK, row, 0)
    pltpu.sync_copy(y1, out_hbm.at[pl.ds(base, CHUNK)])


def kernel(x, Wg1, bg1, Wg2, bg2, W1, b1, W2, b2):
    B, S, _ = x.shape
    x_flat = x.reshape(-1, D)
    pos1, pos2, w1b, w2b, eid = _gate(x_flat, Wg1, bg1, Wg2, bg2)
    pos1f = pos1.reshape(N)
    pos2f = pos2.reshape(N)
    xs, ws = _sc_scatter(x_flat, pos1f, pos2f, w1b, w2b)
    y = _ffn(eid.reshape(G), xs, ws, W1, b1, W2, b2)
    out = _sc_combine(y, pos1f, pos2f)
    return out.reshape(B, S, D)


# speedup vs baseline: 1.0403x; 1.0403x over previous
import functools

import jax
import jax.numpy as jnp
from jax import lax
from jax.experimental import pallas as pl
from jax.experimental.pallas import tpu as pltpu
from jax.experimental.pallas import tpu_sc as plsc

N = 2048
D = 768
F = 1536
E = 8
TB = 128
G = 39
P = G * TB
NW = 32
CHUNK = N // NW


def _gate_body(x_ref, wg1_ref, bg1_ref, wg2_ref, bg2_ref,
               pos1_ref, pos2_ref, w1b_ref, w2b_ref, eid_ref):
    xb = x_ref[...]
    g1 = jnp.maximum(
        jnp.dot(xb, wg1_ref[...], preferred_element_type=jnp.float32)
        + bg1_ref[...], 0.0)
    logits = (jnp.dot(g1, wg2_ref[...], preferred_element_type=jnp.float32)
              + bg2_ref[...])
    m = jnp.max(logits, axis=-1, keepdims=True)
    ex = jnp.exp(logits - m)
    p = ex / jnp.sum(ex, axis=-1, keepdims=True)
    iota_e = jax.lax.broadcasted_iota(jnp.int32, (N, E), 1)
    w1 = jnp.max(p, axis=-1, keepdims=True)
    i1 = jnp.min(jnp.where(p >= w1, iota_e, E + 1), axis=-1, keepdims=True)
    pm = jnp.where(iota_e == i1, -jnp.inf, p)
    w2 = jnp.max(pm, axis=-1, keepdims=True)
    i2 = jnp.min(jnp.where(pm >= w2, iota_e, E + 1), axis=-1, keepdims=True)
    a = jnp.exp(w2 - w1)
    c1 = 1.0 / (1.0 + a)
    c2 = a * c1

    oh1 = (iota_e == i1).astype(jnp.float32)
    oh2 = (iota_e == i2).astype(jnp.float32)
    slot = oh1 + oh2
    rr = jax.lax.broadcasted_iota(jnp.int32, (N, N), 0)
    cc = jax.lax.broadcasted_iota(jnp.int32, (N, N), 1)
    lt = (cc < rr).astype(jnp.float32)
    rank = jnp.dot(lt, slot, preferred_element_type=jnp.float32)
    counts = jnp.sum(slot, axis=0, keepdims=True)
    cblk = jnp.floor((counts + (TB - 1.0)) * (1.0 / TB))
    ei = jax.lax.broadcasted_iota(jnp.int32, (E, E), 0)
    ej = jax.lax.broadcasted_iota(jnp.int32, (E, E), 1)
    ltE = (ei < ej).astype(jnp.float32)
    off_al = jnp.dot(cblk, ltE, preferred_element_type=jnp.float32) * TB
    dest = off_al + rank
    pos1_ref[...] = jnp.sum(oh1 * dest, axis=-1, keepdims=True).astype(jnp.int32)
    pos2_ref[...] = jnp.sum(oh2 * dest, axis=-1, keepdims=True).astype(jnp.int32)
    w1b_ref[...] = jnp.broadcast_to(c1, (N, 16))
    w2b_ref[...] = jnp.broadcast_to(c2, (N, 16))
    gi = jax.lax.broadcasted_iota(jnp.int32, (G, E), 0).astype(jnp.float32) * TB
    started = (jnp.broadcast_to(off_al, (G, E)) <= gi).astype(jnp.float32)
    eid_ref[...] = (jnp.sum(started, axis=-1, keepdims=True) - 1.0).astype(jnp.int32)


def _gate(x_flat, Wg1, bg1, Wg2, bg2):
    return pl.pallas_call(
        _gate_body,
        in_specs=[
            pl.BlockSpec((N, D), lambda: (0, 0)),
            pl.BlockSpec((D, D // 2), lambda: (0, 0)),
            pl.BlockSpec((1, D // 2), lambda: (0, 0)),
            pl.BlockSpec((D // 2, E), lambda: (0, 0)),
            pl.BlockSpec((1, E), lambda: (0, 0)),
        ],
        out_specs=[
            pl.BlockSpec((N, 1), lambda: (0, 0)),
            pl.BlockSpec((N, 1), lambda: (0, 0)),
            pl.BlockSpec((N, 16), lambda: (0, 0)),
            pl.BlockSpec((N, 16), lambda: (0, 0)),
            pl.BlockSpec((G, 1), lambda: (0, 0)),
        ],
        out_shape=[
            jax.ShapeDtypeStruct((N, 1), jnp.int32),
            jax.ShapeDtypeStruct((N, 1), jnp.int32),
            jax.ShapeDtypeStruct((N, 16), jnp.float32),
            jax.ShapeDtypeStruct((N, 16), jnp.float32),
            jax.ShapeDtypeStruct((G, 1), jnp.int32),
        ],
    )(x_flat, Wg1, bg1.reshape(1, -1), Wg2, bg2.reshape(1, -1))


_SC_MESH = plsc.VectorSubcoreMesh(core_axis_name="c", subcore_axis_name="s")


@functools.partial(
    pl.kernel,
    out_type=jax.ShapeDtypeStruct((P, D), jnp.float32),
    mesh=_SC_MESH,
    scratch_types=[
        pltpu.VMEM((CHUNK, D), jnp.float32),
        pltpu.VMEM((CHUNK,), jnp.int32),
        pltpu.VMEM((CHUNK,), jnp.int32),
        pltpu.SemaphoreType.DMA,
        pltpu.SemaphoreType.DMA,
    ],
)
def _sc_scatter(x_hbm, pos1_hbm, pos2_hbm, xs_hbm, xbuf, i1v, i2v, sem, xsem):
    wid = lax.axis_index("s") * 2 + lax.axis_index("c")
    base = wid * CHUNK
    cx = pltpu.async_copy(x_hbm.at[pl.ds(base, CHUNK)], xbuf, xsem)
    pltpu.sync_copy(pos1_hbm.at[pl.ds(base, CHUNK)], i1v)
    pltpu.sync_copy(pos2_hbm.at[pl.ds(base, CHUNK)], i2v)
    cx.wait()
    c1 = pltpu.async_copy(xbuf, xs_hbm.at[i1v], sem)
    c2 = pltpu.async_copy(xbuf, xs_hbm.at[i2v], sem)
    c1.wait()
    c2.wait()


def _ffn_body(eid_ref, x_ref, w1_ref, b1_ref, w2_ref, b2_ref, o_ref):
    del eid_ref
    xb = x_ref[...]
    h = (jnp.dot(xb, w1_ref[0], preferred_element_type=jnp.float32)
         + b1_ref[0])
    h = 0.5 * h * (1.0 + jax.lax.erf(h * (2.0 ** -0.5)))
    o_ref[...] = (jnp.dot(h, w2_ref[0], preferred_element_type=jnp.float32)
                  + b2_ref[0])


def _ffn(eid, xs, W1, b1, W2, b2):
    grid_spec = pltpu.PrefetchScalarGridSpec(
        num_scalar_prefetch=1,
        grid=(G,),
        in_specs=[
            pl.BlockSpec((TB, D), lambda g, eid: (g, 0)),
            pl.BlockSpec((1, D, F), lambda g, eid: (eid[g], 0, 0)),
            pl.BlockSpec((1, 1, F), lambda g, eid: (eid[g], 0, 0)),
            pl.BlockSpec((1, F, D), lambda g, eid: (eid[g], 0, 0)),
            pl.BlockSpec((1, 1, D), lambda g, eid: (eid[g], 0, 0)),
        ],
        out_specs=pl.BlockSpec((TB, D), lambda g, eid: (g, 0)),
    )
    return pl.pallas_call(
        _ffn_body,
        grid_spec=grid_spec,
        out_shape=jax.ShapeDtypeStruct((P, D), jnp.float32),
    )(eid, xs, W1, b1.reshape(E, 1, F), W2, b2.reshape(E, 1, D))


@functools.partial(
    pl.kernel,
    out_type=jax.ShapeDtypeStruct((N, D), jnp.float32),
    mesh=_SC_MESH,
    scratch_types=[
        pltpu.VMEM((CHUNK, D), jnp.float32),
        pltpu.VMEM((CHUNK, D), jnp.float32),
        pltpu.VMEM((CHUNK,), jnp.int32),
        pltpu.VMEM((CHUNK,), jnp.int32),
        pltpu.VMEM((CHUNK, 16), jnp.float32),
        pltpu.VMEM((CHUNK, 16), jnp.float32),
        pltpu.SemaphoreType.DMA,
        pltpu.SemaphoreType.DMA,
    ],
)
def _sc_combine(y_hbm, pos1_hbm, pos2_hbm, w1b_hbm, w2b_hbm, out_hbm,
                y1, y2, i1v, i2v, w1v, w2v, sem, wsem):
    wid = lax.axis_index("s") * 2 + lax.axis_index("c")
    base = wid * CHUNK
    pltpu.sync_copy(pos1_hbm.at[pl.ds(base, CHUNK)], i1v)
    pltpu.sync_copy(pos2_hbm.at[pl.ds(base, CHUNK)], i2v)
    c1 = pltpu.async_copy(y_hbm.at[i1v], y1, sem)
    c2 = pltpu.async_copy(y_hbm.at[i2v], y2, sem)
    cw1 = pltpu.async_copy(w1b_hbm.at[pl.ds(base, CHUNK)], w1v, wsem)
    cw2 = pltpu.async_copy(w2b_hbm.at[pl.ds(base, CHUNK)], w2v, wsem)
    c1.wait()
    c2.wait()
    cw1.wait()
    cw2.wait()

    def row(i, _):
        for j in range(D // 16):
            sl = pl.ds(j * 16, 16)
            y1[i, sl] = y1[i, sl] * w1v[i, :] + y2[i, sl] * w2v[i, :]
        return 0

    lax.fori_loop(0, CHUNK, row, 0)
    pltpu.sync_copy(y1, out_hbm.at[pl.ds(base, CHUNK)])


def kernel(x, Wg1, bg1, Wg2, bg2, W1, b1, W2, b2):
    B, S, _ = x.shape
    x_flat = x.reshape(-1, D)
    pos1, pos2, w1b, w2b, eid = _gate(x_flat, Wg1, bg1, Wg2, bg2)
    pos1f = pos1.reshape(N)
    pos2f = pos2.reshape(N)
    xs = _sc_scatter(x_flat, pos1f, pos2f)
    y = _ffn(eid.reshape(G), xs, W1, b1, W2, b2)
    out = _sc_combine(y, pos1f, pos2f, w1b, w2b)
    return out.reshape(B, S, D)


# speedup vs baseline: 1.1062x; 1.0634x over previous
import functools

import jax
import jax.numpy as jnp
from jax import lax
from jax.experimental import pallas as pl
from jax.experimental.pallas import tpu as pltpu
from jax.experimental.pallas import tpu_sc as plsc

N = 2048
D = 768
F = 1536
E = 8
TB = 256
G = 24
P = G * TB
NW = 32
CHUNK = N // NW


def _gate_body(x_ref, wg1_ref, bg1_ref, wg2_ref, bg2_ref,
               pos1_ref, pos2_ref, w1b_ref, w2b_ref, eid_ref):
    xb = x_ref[...]
    g1 = jnp.maximum(
        jnp.dot(xb, wg1_ref[...], preferred_element_type=jnp.float32)
        + bg1_ref[...], 0.0)
    logits = (jnp.dot(g1, wg2_ref[...], preferred_element_type=jnp.float32)
              + bg2_ref[...])
    m = jnp.max(logits, axis=-1, keepdims=True)
    ex = jnp.exp(logits - m)
    p = ex / jnp.sum(ex, axis=-1, keepdims=True)
    iota_e = jax.lax.broadcasted_iota(jnp.int32, (N, E), 1)
    w1 = jnp.max(p, axis=-1, keepdims=True)
    i1 = jnp.min(jnp.where(p >= w1, iota_e, E + 1), axis=-1, keepdims=True)
    pm = jnp.where(iota_e == i1, -jnp.inf, p)
    w2 = jnp.max(pm, axis=-1, keepdims=True)
    i2 = jnp.min(jnp.where(pm >= w2, iota_e, E + 1), axis=-1, keepdims=True)
    a = jnp.exp(w2 - w1)
    c1 = 1.0 / (1.0 + a)
    c2 = a * c1

    oh1 = (iota_e == i1).astype(jnp.float32)
    oh2 = (iota_e == i2).astype(jnp.float32)
    slot = oh1 + oh2
    rr = jax.lax.broadcasted_iota(jnp.int32, (N, N), 0)
    cc = jax.lax.broadcasted_iota(jnp.int32, (N, N), 1)
    lt = (cc < rr).astype(jnp.float32)
    rank = jnp.dot(lt, slot, preferred_element_type=jnp.float32)
    counts = jnp.sum(slot, axis=0, keepdims=True)
    cblk = jnp.floor((counts + (TB - 1.0)) * (1.0 / TB))
    ei = jax.lax.broadcasted_iota(jnp.int32, (E, E), 0)
    ej = jax.lax.broadcasted_iota(jnp.int32, (E, E), 1)
    ltE = (ei < ej).astype(jnp.float32)
    off_al = jnp.dot(cblk, ltE, preferred_element_type=jnp.float32) * TB
    dest = off_al + rank
    pos1_ref[...] = jnp.sum(oh1 * dest, axis=-1, keepdims=True).astype(jnp.int32)
    pos2_ref[...] = jnp.sum(oh2 * dest, axis=-1, keepdims=True).astype(jnp.int32)
    w1b_ref[...] = jnp.broadcast_to(c1, (N, 16))
    w2b_ref[...] = jnp.broadcast_to(c2, (N, 16))
    gi = jax.lax.broadcasted_iota(jnp.int32, (G, E), 0).astype(jnp.float32) * TB
    started = (jnp.broadcast_to(off_al, (G, E)) <= gi).astype(jnp.float32)
    eid_ref[...] = (jnp.sum(started, axis=-1, keepdims=True) - 1.0).astype(jnp.int32)


def _gate(x_flat, Wg1, bg1, Wg2, bg2):
    return pl.pallas_call(
        _gate_body,
        in_specs=[
            pl.BlockSpec((N, D), lambda: (0, 0)),
            pl.BlockSpec((D, D // 2), lambda: (0, 0)),
            pl.BlockSpec((1, D // 2), lambda: (0, 0)),
            pl.BlockSpec((D // 2, E), lambda: (0, 0)),
            pl.BlockSpec((1, E), lambda: (0, 0)),
        ],
        out_specs=[
            pl.BlockSpec((N, 1), lambda: (0, 0)),
            pl.BlockSpec((N, 1), lambda: (0, 0)),
            pl.BlockSpec((N, 16), lambda: (0, 0)),
            pl.BlockSpec((N, 16), lambda: (0, 0)),
            pl.BlockSpec((G, 1), lambda: (0, 0)),
        ],
        out_shape=[
            jax.ShapeDtypeStruct((N, 1), jnp.int32),
            jax.ShapeDtypeStruct((N, 1), jnp.int32),
            jax.ShapeDtypeStruct((N, 16), jnp.float32),
            jax.ShapeDtypeStruct((N, 16), jnp.float32),
            jax.ShapeDtypeStruct((G, 1), jnp.int32),
        ],
    )(x_flat, Wg1, bg1.reshape(1, -1), Wg2, bg2.reshape(1, -1))


_SC_MESH = plsc.VectorSubcoreMesh(core_axis_name="c", subcore_axis_name="s")


@functools.partial(
    pl.kernel,
    out_type=jax.ShapeDtypeStruct((P, D), jnp.float32),
    mesh=_SC_MESH,
    scratch_types=[
        pltpu.VMEM((CHUNK, D), jnp.float32),
        pltpu.VMEM((CHUNK,), jnp.int32),
        pltpu.VMEM((CHUNK,), jnp.int32),
        pltpu.SemaphoreType.DMA,
        pltpu.SemaphoreType.DMA,
    ],
)
def _sc_scatter(x_hbm, pos1_hbm, pos2_hbm, xs_hbm, xbuf, i1v, i2v, sem, xsem):
    wid = lax.axis_index("s") * 2 + lax.axis_index("c")
    base = wid * CHUNK
    cx = pltpu.async_copy(x_hbm.at[pl.ds(base, CHUNK)], xbuf, xsem)
    pltpu.sync_copy(pos1_hbm.at[pl.ds(base, CHUNK)], i1v)
    pltpu.sync_copy(pos2_hbm.at[pl.ds(base, CHUNK)], i2v)
    cx.wait()
    c1 = pltpu.async_copy(xbuf, xs_hbm.at[i1v], sem)
    c2 = pltpu.async_copy(xbuf, xs_hbm.at[i2v], sem)
    c1.wait()
    c2.wait()


def _ffn_body(eid_ref, x_ref, w1_ref, b1_ref, w2_ref, b2_ref, o_ref):
    del eid_ref
    xb = x_ref[...]
    h = (jnp.dot(xb, w1_ref[0], preferred_element_type=jnp.float32)
         + b1_ref[0])
    h = 0.5 * h * (1.0 + jax.lax.erf(h * (2.0 ** -0.5)))
    o_ref[...] = (jnp.dot(h, w2_ref[0], preferred_element_type=jnp.float32)
                  + b2_ref[0])


def _ffn(eid, xs, W1, b1, W2, b2):
    grid_spec = pltpu.PrefetchScalarGridSpec(
        num_scalar_prefetch=1,
        grid=(G,),
        in_specs=[
            pl.BlockSpec((TB, D), lambda g, eid: (g, 0)),
            pl.BlockSpec((1, D, F), lambda g, eid: (eid[g], 0, 0)),
            pl.BlockSpec((1, 1, F), lambda g, eid: (eid[g], 0, 0)),
            pl.BlockSpec((1, F, D), lambda g, eid: (eid[g], 0, 0)),
            pl.BlockSpec((1, 1, D), lambda g, eid: (eid[g], 0, 0)),
        ],
        out_specs=pl.BlockSpec((TB, D), lambda g, eid: (g, 0)),
    )
    return pl.pallas_call(
        _ffn_body,
        grid_spec=grid_spec,
        out_shape=jax.ShapeDtypeStruct((P, D), jnp.float32),
    )(eid, xs, W1, b1.reshape(E, 1, F), W2, b2.reshape(E, 1, D))


@functools.partial(
    pl.kernel,
    out_type=jax.ShapeDtypeStruct((N, D), jnp.float32),
    mesh=_SC_MESH,
    scratch_types=[
        pltpu.VMEM((CHUNK, D), jnp.float32),
        pltpu.VMEM((CHUNK, D), jnp.float32),
        pltpu.VMEM((CHUNK,), jnp.int32),
        pltpu.VMEM((CHUNK,), jnp.int32),
        pltpu.VMEM((CHUNK, 16), jnp.float32),
        pltpu.VMEM((CHUNK, 16), jnp.float32),
        pltpu.SemaphoreType.DMA,
        pltpu.SemaphoreType.DMA,
    ],
)
def _sc_combine(y_hbm, pos1_hbm, pos2_hbm, w1b_hbm, w2b_hbm, out_hbm,
                y1, y2, i1v, i2v, w1v, w2v, sem, wsem):
    wid = lax.axis_index("s") * 2 + lax.axis_index("c")
    base = wid * CHUNK
    pltpu.sync_copy(pos1_hbm.at[pl.ds(base, CHUNK)], i1v)
    pltpu.sync_copy(pos2_hbm.at[pl.ds(base, CHUNK)], i2v)
    c1 = pltpu.async_copy(y_hbm.at[i1v], y1, sem)
    c2 = pltpu.async_copy(y_hbm.at[i2v], y2, sem)
    cw1 = pltpu.async_copy(w1b_hbm.at[pl.ds(base, CHUNK)], w1v, wsem)
    cw2 = pltpu.async_copy(w2b_hbm.at[pl.ds(base, CHUNK)], w2v, wsem)
    c1.wait()
    c2.wait()
    cw1.wait()
    cw2.wait()

    def row(i, _):
        for j in range(D // 16):
            sl = pl.ds(j * 16, 16)
            y1[i, sl] = y1[i, sl] * w1v[i, :] + y2[i, sl] * w2v[i, :]
        return 0

    lax.fori_loop(0, CHUNK, row, 0)
    pltpu.sync_copy(y1, out_hbm.at[pl.ds(base, CHUNK)])


def kernel(x, Wg1, bg1, Wg2, bg2, W1, b1, W2, b2):
    B, S, _ = x.shape
    x_flat = x.reshape(-1, D)
    pos1, pos2, w1b, w2b, eid = _gate(x_flat, Wg1, bg1, Wg2, bg2)
    pos1f = pos1.reshape(N)
    pos2f = pos2.reshape(N)
    xs = _sc_scatter(x_flat, pos1f, pos2f)
    y = _ffn(eid.reshape(G), xs, W1, b1, W2, b2)
    out = _sc_combine(y, pos1f, pos2f, w1b, w2b)
    return out.reshape(B, S, D)


# speedup vs baseline: 1.1178x; 1.0104x over previous
import functools

import jax
import jax.numpy as jnp
from jax import lax
from jax.experimental import pallas as pl
from jax.experimental.pallas import tpu as pltpu
from jax.experimental.pallas import tpu_sc as plsc

N = 2048
D = 768
F = 1536
E = 8
TB = 256
G = 23
P = G * TB
NW = 32
CHUNK = N // NW


def _gate_body(x_ref, wg1_ref, bg1_ref, wg2_ref, bg2_ref,
               pos1_ref, pos2_ref, w1b_ref, w2b_ref, eid_ref):
    xb = x_ref[...]
    g1 = jnp.maximum(
        jnp.dot(xb, wg1_ref[...], preferred_element_type=jnp.float32)
        + bg1_ref[...], 0.0)
    logits = (jnp.dot(g1, wg2_ref[...], preferred_element_type=jnp.float32)
              + bg2_ref[...])
    m = jnp.max(logits, axis=-1, keepdims=True)
    ex = jnp.exp(logits - m)
    p = ex / jnp.sum(ex, axis=-1, keepdims=True)
    iota_e = jax.lax.broadcasted_iota(jnp.int32, (N, E), 1)
    w1 = jnp.max(p, axis=-1, keepdims=True)
    i1 = jnp.min(jnp.where(p >= w1, iota_e, E + 1), axis=-1, keepdims=True)
    pm = jnp.where(iota_e == i1, -jnp.inf, p)
    w2 = jnp.max(pm, axis=-1, keepdims=True)
    i2 = jnp.min(jnp.where(pm >= w2, iota_e, E + 1), axis=-1, keepdims=True)
    a = jnp.exp(w2 - w1)
    c1 = 1.0 / (1.0 + a)
    c2 = a * c1

    oh1 = (iota_e == i1).astype(jnp.float32)
    oh2 = (iota_e == i2).astype(jnp.float32)
    slot = oh1 + oh2
    rr = jax.lax.broadcasted_iota(jnp.int32, (N, N), 0)
    cc = jax.lax.broadcasted_iota(jnp.int32, (N, N), 1)
    lt = (cc < rr).astype(jnp.float32)
    rank = jnp.dot(lt, slot, preferred_element_type=jnp.float32)
    counts = jnp.sum(slot, axis=0, keepdims=True)
    cblk = jnp.floor((counts + (TB - 1.0)) * (1.0 / TB))
    ei = jax.lax.broadcasted_iota(jnp.int32, (E, E), 0)
    ej = jax.lax.broadcasted_iota(jnp.int32, (E, E), 1)
    ltE = (ei < ej).astype(jnp.float32)
    off_al = jnp.dot(cblk, ltE, preferred_element_type=jnp.float32) * TB
    dest = off_al + rank
    pos1_ref[...] = jnp.sum(oh1 * dest, axis=-1, keepdims=True).astype(jnp.int32)
    pos2_ref[...] = jnp.sum(oh2 * dest, axis=-1, keepdims=True).astype(jnp.int32)
    w1b_ref[...] = jnp.broadcast_to(c1, (N, 16))
    w2b_ref[...] = jnp.broadcast_to(c2, (N, 16))
    gi = jax.lax.broadcasted_iota(jnp.int32, (G, E), 0).astype(jnp.float32) * TB
    started = (jnp.broadcast_to(off_al, (G, E)) <= gi).astype(jnp.float32)
    eid_ref[...] = (jnp.sum(started, axis=-1, keepdims=True) - 1.0).astype(jnp.int32)


def _gate(x_flat, Wg1, bg1, Wg2, bg2):
    return pl.pallas_call(
        _gate_body,
        in_specs=[
            pl.BlockSpec((N, D), lambda: (0, 0)),
            pl.BlockSpec((D, D // 2), lambda: (0, 0)),
            pl.BlockSpec((1, D // 2), lambda: (0, 0)),
            pl.BlockSpec((D // 2, E), lambda: (0, 0)),
            pl.BlockSpec((1, E), lambda: (0, 0)),
        ],
        out_specs=[
            pl.BlockSpec((N, 1), lambda: (0, 0)),
            pl.BlockSpec((N, 1), lambda: (0, 0)),
            pl.BlockSpec((N, 16), lambda: (0, 0)),
            pl.BlockSpec((N, 16), lambda: (0, 0)),
            pl.BlockSpec((G, 1), lambda: (0, 0)),
        ],
        out_shape=[
            jax.ShapeDtypeStruct((N, 1), jnp.int32),
            jax.ShapeDtypeStruct((N, 1), jnp.int32),
            jax.ShapeDtypeStruct((N, 16), jnp.float32),
            jax.ShapeDtypeStruct((N, 16), jnp.float32),
            jax.ShapeDtypeStruct((G, 1), jnp.int32),
        ],
    )(x_flat, Wg1, bg1.reshape(1, -1), Wg2, bg2.reshape(1, -1))


_SC_MESH = plsc.VectorSubcoreMesh(core_axis_name="c", subcore_axis_name="s")


@functools.partial(
    pl.kernel,
    out_type=jax.ShapeDtypeStruct((P, D), jnp.float32),
    mesh=_SC_MESH,
    scratch_types=[
        pltpu.VMEM((CHUNK, D), jnp.float32),
        pltpu.VMEM((CHUNK,), jnp.int32),
        pltpu.VMEM((CHUNK,), jnp.int32),
        pltpu.SemaphoreType.DMA,
        pltpu.SemaphoreType.DMA,
    ],
)
def _sc_scatter(x_hbm, pos1_hbm, pos2_hbm, xs_hbm, xbuf, i1v, i2v, sem, xsem):
    wid = lax.axis_index("s") * 2 + lax.axis_index("c")
    base = wid * CHUNK
    cx = pltpu.async_copy(x_hbm.at[pl.ds(base, CHUNK)], xbuf, xsem)
    pltpu.sync_copy(pos1_hbm.at[pl.ds(base, CHUNK)], i1v)
    pltpu.sync_copy(pos2_hbm.at[pl.ds(base, CHUNK)], i2v)
    cx.wait()
    c1 = pltpu.async_copy(xbuf, xs_hbm.at[i1v], sem)
    c2 = pltpu.async_copy(xbuf, xs_hbm.at[i2v], sem)
    c1.wait()
    c2.wait()


def _ffn_body(eid_ref, x_ref, w1_ref, b1_ref, w2_ref, b2_ref, o_ref):
    del eid_ref
    xb = x_ref[...]
    h = (jnp.dot(xb, w1_ref[0], preferred_element_type=jnp.float32)
         + b1_ref[0])
    h = 0.5 * h * (1.0 + jax.lax.erf(h * (2.0 ** -0.5)))
    o_ref[...] = (jnp.dot(h, w2_ref[0], preferred_element_type=jnp.float32)
                  + b2_ref[0])


def _ffn(eid, xs, W1, b1, W2, b2):
    grid_spec = pltpu.PrefetchScalarGridSpec(
        num_scalar_prefetch=1,
        grid=(G,),
        in_specs=[
            pl.BlockSpec((TB, D), lambda g, eid: (g, 0)),
            pl.BlockSpec((1, D, F), lambda g, eid: (eid[g], 0, 0)),
            pl.BlockSpec((1, 1, F), lambda g, eid: (eid[g], 0, 0)),
            pl.BlockSpec((1, F, D), lambda g, eid: (eid[g], 0, 0)),
            pl.BlockSpec((1, 1, D), lambda g, eid: (eid[g], 0, 0)),
        ],
        out_specs=pl.BlockSpec((TB, D), lambda g, eid: (g, 0)),
    )
    return pl.pallas_call(
        _ffn_body,
        grid_spec=grid_spec,
        out_shape=jax.ShapeDtypeStruct((P, D), jnp.float32),
    )(eid, xs, W1, b1.reshape(E, 1, F), W2, b2.reshape(E, 1, D))


@functools.partial(
    pl.kernel,
    out_type=jax.ShapeDtypeStruct((N, D), jnp.float32),
    mesh=_SC_MESH,
    scratch_types=[
        pltpu.VMEM((CHUNK, D), jnp.float32),
        pltpu.VMEM((CHUNK, D), jnp.float32),
        pltpu.VMEM((CHUNK,), jnp.int32),
        pltpu.VMEM((CHUNK,), jnp.int32),
        pltpu.VMEM((CHUNK, 16), jnp.float32),
        pltpu.VMEM((CHUNK, 16), jnp.float32),
        pltpu.SemaphoreType.DMA,
        pltpu.SemaphoreType.DMA,
    ],
)
def _sc_combine(y_hbm, pos1_hbm, pos2_hbm, w1b_hbm, w2b_hbm, out_hbm,
                y1, y2, i1v, i2v, w1v, w2v, sem, wsem):
    wid = lax.axis_index("s") * 2 + lax.axis_index("c")
    base = wid * CHUNK
    pltpu.sync_copy(pos1_hbm.at[pl.ds(base, CHUNK)], i1v)
    pltpu.sync_copy(pos2_hbm.at[pl.ds(base, CHUNK)], i2v)
    c1 = pltpu.async_copy(y_hbm.at[i1v], y1, sem)
    c2 = pltpu.async_copy(y_hbm.at[i2v], y2, sem)
    cw1 = pltpu.async_copy(w1b_hbm.at[pl.ds(base, CHUNK)], w1v, wsem)
    cw2 = pltpu.async_copy(w2b_hbm.at[pl.ds(base, CHUNK)], w2v, wsem)
    c1.wait()
    c2.wait()
    cw1.wait()
    cw2.wait()

    def row(i, _):
        for j in range(D // 16):
            sl = pl.ds(j * 16, 16)
            y1[i, sl] = y1[i, sl] * w1v[i, :] + y2[i, sl] * w2v[i, :]
        return 0

    lax.fori_loop(0, CHUNK, row, 0)
    pltpu.sync_copy(y1, out_hbm.at[pl.ds(base, CHUNK)])


def kernel(x, Wg1, bg1, Wg2, bg2, W1, b1, W2, b2):
    B, S, _ = x.shape
    x_flat = x.reshape(-1, D)
    pos1, pos2, w1b, w2b, eid = _gate(x_flat, Wg1, bg1, Wg2, bg2)
    pos1f = pos1.reshape(N)
    pos2f = pos2.reshape(N)
    xs = _sc_scatter(x_flat, pos1f, pos2f)
    y = _ffn(eid.reshape(G), xs, W1, b1, W2, b2)
    out = _sc_combine(y, pos1f, pos2f, w1b, w2b)
    return out.reshape(B, S, D)


# speedup vs baseline: 1.2681x; 1.1345x over previous
import functools

import jax
import jax.numpy as jnp
from jax import lax
from jax.experimental import pallas as pl
from jax.experimental.pallas import tpu as pltpu
from jax.experimental.pallas import tpu_sc as plsc

N = 2048
D = 768
F = 1536
E = 8
TB = 256
G = 23
P = G * TB
NW = 32
CHUNK = N // NW


def _gate_body(x_ref, wg1_ref, bg1_ref, wg2_ref, bg2_ref,
               pos1_ref, pos2_ref, w1b_ref, w2b_ref, eid_ref):
    xb = x_ref[...]
    g1 = jnp.maximum(
        jnp.dot(xb, wg1_ref[...], preferred_element_type=jnp.float32)
        + bg1_ref[...], 0.0)
    logits = (jnp.dot(g1, wg2_ref[...], preferred_element_type=jnp.float32)
              + bg2_ref[...])
    m = jnp.max(logits, axis=-1, keepdims=True)
    ex = jnp.exp(logits - m)
    p = ex / jnp.sum(ex, axis=-1, keepdims=True)
    iota_e = jax.lax.broadcasted_iota(jnp.int32, (N, E), 1)
    w1 = jnp.max(p, axis=-1, keepdims=True)
    i1 = jnp.min(jnp.where(p >= w1, iota_e, E + 1), axis=-1, keepdims=True)
    pm = jnp.where(iota_e == i1, -jnp.inf, p)
    w2 = jnp.max(pm, axis=-1, keepdims=True)
    i2 = jnp.min(jnp.where(pm >= w2, iota_e, E + 1), axis=-1, keepdims=True)
    a = jnp.exp(w2 - w1)
    c1 = 1.0 / (1.0 + a)
    c2 = a * c1

    oh1 = (iota_e == i1).astype(jnp.float32)
    oh2 = (iota_e == i2).astype(jnp.float32)
    slot = oh1 + oh2
    rr = jax.lax.broadcasted_iota(jnp.int32, (N, N), 0)
    cc = jax.lax.broadcasted_iota(jnp.int32, (N, N), 1)
    lt = (cc < rr).astype(jnp.float32)
    rank = jnp.dot(lt, slot, preferred_element_type=jnp.float32)
    counts = jnp.sum(slot, axis=0, keepdims=True)
    cblk = jnp.floor((counts + (TB - 1.0)) * (1.0 / TB))
    ei = jax.lax.broadcasted_iota(jnp.int32, (E, E), 0)
    ej = jax.lax.broadcasted_iota(jnp.int32, (E, E), 1)
    ltE = (ei < ej).astype(jnp.float32)
    off_al = jnp.dot(cblk, ltE, preferred_element_type=jnp.float32) * TB
    dest = off_al + rank
    pos1_ref[...] = jnp.sum(oh1 * dest, axis=-1, keepdims=True).astype(jnp.int32)
    pos2_ref[...] = jnp.sum(oh2 * dest, axis=-1, keepdims=True).astype(jnp.int32)
    w1b_ref[...] = jnp.broadcast_to(c1, (N, 16))
    w2b_ref[...] = jnp.broadcast_to(c2, (N, 16))
    gi = jax.lax.broadcasted_iota(jnp.int32, (G, E), 0).astype(jnp.float32) * TB
    started = (jnp.broadcast_to(off_al, (G, E)) <= gi).astype(jnp.float32)
    eid_ref[...] = (jnp.sum(started, axis=-1, keepdims=True) - 1.0).astype(jnp.int32)


def _gate(x_flat, Wg1, bg1, Wg2, bg2):
    return pl.pallas_call(
        _gate_body,
        in_specs=[
            pl.BlockSpec((N, D), lambda: (0, 0)),
            pl.BlockSpec((D, D // 2), lambda: (0, 0)),
            pl.BlockSpec((1, D // 2), lambda: (0, 0)),
            pl.BlockSpec((D // 2, E), lambda: (0, 0)),
            pl.BlockSpec((1, E), lambda: (0, 0)),
        ],
        out_specs=[
            pl.BlockSpec((N, 1), lambda: (0, 0)),
            pl.BlockSpec((N, 1), lambda: (0, 0)),
            pl.BlockSpec((N, 16), lambda: (0, 0)),
            pl.BlockSpec((N, 16), lambda: (0, 0)),
            pl.BlockSpec((G, 1), lambda: (0, 0)),
        ],
        out_shape=[
            jax.ShapeDtypeStruct((N, 1), jnp.int32),
            jax.ShapeDtypeStruct((N, 1), jnp.int32),
            jax.ShapeDtypeStruct((N, 16), jnp.float32),
            jax.ShapeDtypeStruct((N, 16), jnp.float32),
            jax.ShapeDtypeStruct((G, 1), jnp.int32),
        ],
    )(x_flat, Wg1, bg1.reshape(1, -1), Wg2, bg2.reshape(1, -1))


_SC_MESH = plsc.VectorSubcoreMesh(core_axis_name="c", subcore_axis_name="s")


@functools.partial(
    pl.kernel,
    out_type=jax.ShapeDtypeStruct((P, D), jnp.float32),
    mesh=_SC_MESH,
    scratch_types=[
        pltpu.VMEM((CHUNK, D), jnp.float32),
        pltpu.VMEM((CHUNK,), jnp.int32),
        pltpu.VMEM((CHUNK,), jnp.int32),
        pltpu.SemaphoreType.DMA,
        pltpu.SemaphoreType.DMA,
    ],
)
def _sc_scatter(x_hbm, pos1_hbm, pos2_hbm, xs_hbm, xbuf, i1v, i2v, sem, xsem):
    wid = lax.axis_index("s") * 2 + lax.axis_index("c")
    base = wid * CHUNK
    cx = pltpu.async_copy(x_hbm.at[pl.ds(base, CHUNK)], xbuf, xsem)
    pltpu.sync_copy(pos1_hbm.at[pl.ds(base, CHUNK)], i1v)
    pltpu.sync_copy(pos2_hbm.at[pl.ds(base, CHUNK)], i2v)
    cx.wait()
    c1 = pltpu.async_copy(xbuf, xs_hbm.at[i1v], sem)
    c2 = pltpu.async_copy(xbuf, xs_hbm.at[i2v], sem)
    c1.wait()
    c2.wait()


def _ffn_body(eid_ref, x_ref, w1_ref, b1_ref, w2_ref, b2_ref, o_ref):
    del eid_ref
    xb = x_ref[...]
    h = (jnp.dot(xb, w1_ref[0], preferred_element_type=jnp.float32)
         + b1_ref[0])
    h = 0.5 * h * (1.0 + jax.lax.erf(h * (2.0 ** -0.5)))
    o_ref[...] = (jnp.dot(h, w2_ref[0], preferred_element_type=jnp.float32)
                  + b2_ref[0])


def _ffn(eid, xs, W1, b1, W2, b2):
    grid_spec = pltpu.PrefetchScalarGridSpec(
        num_scalar_prefetch=1,
        grid=(G,),
        in_specs=[
            pl.BlockSpec((TB, D), lambda g, eid: (g, 0)),
            pl.BlockSpec((1, D, F), lambda g, eid: (eid[g], 0, 0)),
            pl.BlockSpec((1, 1, F), lambda g, eid: (eid[g], 0, 0)),
            pl.BlockSpec((1, F, D), lambda g, eid: (eid[g], 0, 0)),
            pl.BlockSpec((1, 1, D), lambda g, eid: (eid[g], 0, 0)),
        ],
        out_specs=pl.BlockSpec((TB, D), lambda g, eid: (g, 0)),
    )
    return pl.pallas_call(
        _ffn_body,
        grid_spec=grid_spec,
        out_shape=jax.ShapeDtypeStruct((P, D), jnp.float32),
    )(eid, xs, W1, b1.reshape(E, 1, F), W2, b2.reshape(E, 1, D))


@functools.partial(
    pl.kernel,
    out_type=jax.ShapeDtypeStruct((N, D), jnp.float32),
    mesh=_SC_MESH,
    scratch_types=[
        pltpu.VMEM((CHUNK, D), jnp.float32),
        pltpu.VMEM((CHUNK, D), jnp.float32),
        pltpu.VMEM((CHUNK,), jnp.int32),
        pltpu.VMEM((CHUNK,), jnp.int32),
        pltpu.VMEM((CHUNK, 16), jnp.float32),
        pltpu.VMEM((CHUNK, 16), jnp.float32),
        pltpu.SemaphoreType.DMA,
        pltpu.SemaphoreType.DMA,
    ],
)
def _sc_combine(y_hbm, pos1_hbm, pos2_hbm, w1b_hbm, w2b_hbm, out_hbm,
                y1, y2, i1v, i2v, w1v, w2v, sem, wsem):
    wid = lax.axis_index("s") * 2 + lax.axis_index("c")
    base = wid * CHUNK
    pltpu.sync_copy(pos1_hbm.at[pl.ds(base, CHUNK)], i1v)
    pltpu.sync_copy(pos2_hbm.at[pl.ds(base, CHUNK)], i2v)
    c1 = pltpu.async_copy(y_hbm.at[i1v], y1, sem)
    c2 = pltpu.async_copy(y_hbm.at[i2v], y2, sem)
    cw1 = pltpu.async_copy(w1b_hbm.at[pl.ds(base, CHUNK)], w1v, wsem)
    cw2 = pltpu.async_copy(w2b_hbm.at[pl.ds(base, CHUNK)], w2v, wsem)
    c1.wait()
    c2.wait()
    cw1.wait()
    cw2.wait()

    def row(i, _):
        w1 = w1v[i, :]
        w2 = w2v[i, :]
        for j in range(D // 16):
            sl = pl.ds(j * 16, 16)
            y1[i, sl] = y1[i, sl] * w1 + y2[i, sl] * w2
        return 0

    lax.fori_loop(0, CHUNK, row, 0)
    pltpu.sync_copy(y1, out_hbm.at[pl.ds(base, CHUNK)])


def kernel(x, Wg1, bg1, Wg2, bg2, W1, b1, W2, b2):
    B, S, _ = x.shape
    x_flat = x.reshape(-1, D)
    pos1, pos2, w1b, w2b, eid = _gate(x_flat, Wg1, bg1, Wg2, bg2)
    pos1f = pos1.reshape(N)
    pos2f = pos2.reshape(N)
    xs = _sc_scatter(x_flat, pos1f, pos2f)
    y = _ffn(eid.reshape(G), xs, W1, b1, W2, b2)
    out = _sc_combine(y, pos1f, pos2f, w1b, w2b)
    return out.reshape(B, S, D)
